# baseline (speedup 1.0000x reference)
import functools

import jax
import jax.numpy as jnp
from jax import lax
from jax.experimental import pallas as pl
from jax.experimental.pallas import tpu as pltpu
from jax.experimental.pallas import tpu_sc as plsc


def _proj_body(raw_ref, g_ref, c_ref, out_ref):
    out_ref[...] = (
        jnp.dot(raw_ref[...], g_ref[0], preferred_element_type=jnp.float32,
                precision=jax.lax.Precision.HIGHEST)
        + c_ref[0]
    )


def _project(raw_all, G, c, block_rows=1024):
    rows, f = raw_all.shape
    k = G.shape[2]
    n_blocks = rows // block_rows
    per_mod = n_blocks // 2
    return pl.pallas_call(
        _proj_body,
        grid=(n_blocks,),
        in_specs=[
            pl.BlockSpec((block_rows, f), lambda i: (i, 0)),
            pl.BlockSpec((1, f, k), lambda i: (i // per_mod, 0, 0)),
            pl.BlockSpec((1, 1, k), lambda i: (i // per_mod, 0, 0)),
        ],
        out_specs=pl.BlockSpec((block_rows, k), lambda i: (i, 0)),
        out_shape=jax.ShapeDtypeStruct((rows, k), jnp.float32),
    )(raw_all, G, c)


def _make_merge(B, N, L0, L1, K, chunk=128):
    NC, NS = 2, 16
    NW = NC * NS
    wpb = NW // B
    n_per_w = N // wpb
    n_chunks = n_per_w // chunk
    M0 = B * L0
    mesh = plsc.VectorSubcoreMesh(
        core_axis_name="c", subcore_axis_name="s",
        num_cores=NC, num_subcores=NS)

    @functools.partial(
        pl.kernel,
        mesh=mesh,
        out_type=jax.ShapeDtypeStruct((N * B, K), jnp.float32),
        scratch_types=[
            pltpu.VMEM((chunk,), jnp.int32),
            pltpu.VMEM((chunk,), jnp.int32),
            pltpu.VMEM((chunk,), jnp.int32),
            pltpu.VMEM((chunk,), jnp.int32),
            pltpu.VMEM((chunk, K), jnp.float32),
            pltpu.SemaphoreType.DMA,
            pltpu.SemaphoreType.DMA,
        ],
    )
    def merge(table_hbm, e_hbm, m_hbm, out_hbm,
              e_v, m_v, idx_v, oidx_v, rows_v, sem_g, sem_s):
        wid = lax.axis_index("c") * NS + lax.axis_index("s")
        b = wid // wpb
        q = wid % wpb
        iota = lax.iota(jnp.int32, 16)
        for ci in range(n_chunks):
            n0 = q * n_per_w + ci * chunk
            pltpu.sync_copy(e_hbm.at[pl.ds(b * N + n0, chunk)], e_v)
            pltpu.sync_copy(m_hbm.at[pl.ds(b * N + n0, chunk)], m_v)
            for j in range(chunk // 16):
                sl = pl.ds(j * 16, 16)
                ev = e_v[sl]
                mv = m_v[sl]
                idx_v[sl] = mv * M0 + b * L0 + ev
                oidx_v[sl] = (n0 + j * 16 + iota) * B + b
            pltpu.async_copy(table_hbm.at[idx_v], rows_v, sem_g).wait()
            pltpu.async_copy(rows_v, out_hbm.at[oidx_v], sem_s).wait()

    return merge


def _gru_body(gi_ref, wh_ref, bh_ref, len_ref, out_ref, h_sc,
              *, tblk, H, S, nch):
    i = pl.program_id(0)

    @pl.when(i == 0)
    def _init():
        h_sc[...] = jnp.zeros_like(h_sc)
        out_ref[...] = jnp.zeros_like(out_ref)

    wh = wh_ref[...]
    bh = bh_ref[...]
    B = len_ref.shape[0]
    g = B // nch
    tgt = [len_ref[c * g:(c + 1) * g, :] - 1 for c in range(nch)]

    def step(j, carry):
        gi = gi_ref[j]
        t = i * tblk + j
        new = []
        for c in range(nch):
            h, acc = carry[c]
            gic = gi[c * g:(c + 1) * g, :]
            gh = jnp.dot(h, wh, preferred_element_type=jnp.float32,
                         precision=jax.lax.Precision.DEFAULT) + bh
            r = 1.0 / (1.0 + jnp.exp(-(gic[:, 0:H] + gh[:, 0:H])))
            z = 1.0 / (1.0 + jnp.exp(-(gic[:, S:S + H] + gh[:, S:S + H])))
            n = jnp.tanh(gic[:, 2 * S:2 * S + H] + r * gh[:, 2 * S:2 * S + H])
            h2 = (1.0 - z) * n + z * h
            acc2 = jnp.where(tgt[c] == t, h2, acc)
            new.append((h2, acc2))
        return tuple(new)

    carry0 = tuple((h_sc[c * g:(c + 1) * g, :], out_ref[c * g:(c + 1) * g, :])
                   for c in range(nch))
    res = lax.fori_loop(0, tblk, step, carry0)
    for c in range(nch):
        h_sc[c * g:(c + 1) * g, :] = res[c][0]
        out_ref[c * g:(c + 1) * g, :] = res[c][1]


def _gru_scan(gi, Wh, bh, length, tblk=256, nch=2):
    N, B, K = gi.shape
    H = Wh.shape[0]
    nblk = N // tblk
    body = functools.partial(_gru_body, tblk=tblk, H=H, S=K // 3, nch=nch)
    return pl.pallas_call(
        body,
        grid=(nblk,),
        in_specs=[
            pl.BlockSpec((tblk, B, K), lambda i: (i, 0, 0)),
            pl.BlockSpec((H, K), lambda i: (0, 0)),
            pl.BlockSpec((1, K), lambda i: (0, 0)),
            pl.BlockSpec((B, 1), lambda i: (0, 0)),
        ],
        out_specs=pl.BlockSpec((B, H), lambda i: (0, 0)),
        out_shape=jax.ShapeDtypeStruct((B, H), jnp.float32),
        scratch_shapes=[pltpu.VMEM((B, H), jnp.float32)],
    )(gi, Wh, bh, length)


def kernel(raw_0, raw_1, W0, b0, W1, b1, Wi, Wh, bi, bh, time_index,
           seq_lens_0, seq_lens_1):
    B, L0, F = raw_0.shape
    _, L1, _ = raw_1.shape
    N = time_index.shape[1]
    H = Wh.shape[0]
    S = 128
    KP = 3 * S

    def pad_gates(w):
        parts = jnp.split(w, 3, axis=-1)
        pad = [(0, 0)] * (w.ndim - 1) + [(0, S - H)]
        return jnp.concatenate([jnp.pad(p, pad) for p in parts], axis=-1)

    G = pad_gates(jnp.stack([W0 @ Wi, W1 @ Wi]))
    c = pad_gates(jnp.stack([b0 @ Wi, b1 @ Wi]) + bi)[:, None, :]

    raw_all = jnp.concatenate(
        [raw_0.reshape(B * L0, F), raw_1.reshape(B * L1, F)], axis=0)
    gi_all = _project(raw_all, G, c)

    e_flat = time_index[:, :, 0].reshape(-1)
    m_flat = time_index[:, :, 1].reshape(-1)
    merge = _make_merge(B, N, L0, L1, KP)
    gi_t = merge(gi_all, e_flat, m_flat).reshape(N, B, KP)

    length = (seq_lens_0 + seq_lens_1).astype(jnp.int32).reshape(B, 1)
    out = _gru_scan(gi_t, pad_gates(Wh), pad_gates(bh).reshape(1, KP), length)
    return out


# speedup vs baseline: 1.1672x; 1.1672x over previous
import functools

import jax
import jax.numpy as jnp
from jax import lax
from jax.experimental import pallas as pl
from jax.experimental.pallas import tpu as pltpu
from jax.experimental.pallas import tpu_sc as plsc


def _proj_body(raw_ref, g_ref, c_ref, out_ref):
    out_ref[...] = (
        jnp.dot(raw_ref[...], g_ref[0], preferred_element_type=jnp.float32,
                precision=jax.lax.Precision.DEFAULT)
        + c_ref[0]
    )


def _project(raw_all, G, c, block_rows=1024):
    rows, f = raw_all.shape
    k = G.shape[2]
    n_blocks = rows // block_rows
    per_mod = n_blocks // 2
    return pl.pallas_call(
        _proj_body,
        grid=(n_blocks,),
        in_specs=[
            pl.BlockSpec((block_rows, f), lambda i: (i, 0)),
            pl.BlockSpec((1, f, k), lambda i: (i // per_mod, 0, 0)),
            pl.BlockSpec((1, 1, k), lambda i: (i // per_mod, 0, 0)),
        ],
        out_specs=pl.BlockSpec((block_rows, k), lambda i: (i, 0)),
        out_shape=jax.ShapeDtypeStruct((rows, k), jnp.float32),
    )(raw_all, G, c)


def _make_merge(B, N, L0, L1, K, chunk=128):
    NC, NS = 2, 16
    NW = NC * NS
    wpb = NW // B
    n_per_w = N // wpb
    n_chunks = n_per_w // chunk
    M0 = B * L0
    mesh = plsc.VectorSubcoreMesh(
        core_axis_name="c", subcore_axis_name="s",
        num_cores=NC, num_subcores=NS)

    @functools.partial(
        pl.kernel,
        mesh=mesh,
        out_type=jax.ShapeDtypeStruct((N * B, K), jnp.float32),
        scratch_types=[
            pltpu.VMEM((chunk,), jnp.int32),
            pltpu.VMEM((chunk,), jnp.int32),
            pltpu.VMEM((chunk,), jnp.int32),
            pltpu.VMEM((chunk,), jnp.int32),
            pltpu.VMEM((chunk, K), jnp.float32),
            pltpu.SemaphoreType.DMA,
            pltpu.SemaphoreType.DMA,
        ],
    )
    def merge(table_hbm, e_hbm, m_hbm, out_hbm,
              e_v, m_v, idx_v, oidx_v, rows_v, sem_g, sem_s):
        wid = lax.axis_index("c") * NS + lax.axis_index("s")
        b = wid // wpb
        q = wid % wpb
        iota = lax.iota(jnp.int32, 16)
        for ci in range(n_chunks):
            n0 = q * n_per_w + ci * chunk
            pltpu.sync_copy(e_hbm.at[pl.ds(b * N + n0, chunk)], e_v)
            pltpu.sync_copy(m_hbm.at[pl.ds(b * N + n0, chunk)], m_v)
            for j in range(chunk // 16):
                sl = pl.ds(j * 16, 16)
                ev = e_v[sl]
                mv = m_v[sl]
                idx_v[sl] = mv * M0 + b * L0 + ev
                oidx_v[sl] = (n0 + j * 16 + iota) * B + b
            pltpu.async_copy(table_hbm.at[idx_v], rows_v, sem_g).wait()
            pltpu.async_copy(rows_v, out_hbm.at[oidx_v], sem_s).wait()

    return merge


def _gru_body(gi_ref, wh_ref, bh_ref, len_ref, ml_ref, out_ref, h_sc,
              *, tblk, H, S):
    i = pl.program_id(0)

    @pl.when(i == 0)
    def _init():
        h_sc[...] = jnp.zeros_like(h_sc)
        out_ref[...] = jnp.zeros_like(out_ref)

    wh = wh_ref[...]
    bh = bh_ref[...]
    tgt = len_ref[...] - 1

    def step(j, carry):
        h, acc = carry
        gi = gi_ref[j]
        gh = jnp.dot(h, wh, preferred_element_type=jnp.float32,
                     precision=jax.lax.Precision.DEFAULT) + bh
        r = 1.0 / (1.0 + jnp.exp(-(gi[:, 0:H] + gh[:, 0:H])))
        z = 1.0 / (1.0 + jnp.exp(-(gi[:, S:S + H] + gh[:, S:S + H])))
        n = jnp.tanh(gi[:, 2 * S:2 * S + H] + r * gh[:, 2 * S:2 * S + H])
        h2 = (1.0 - z) * n + z * h
        t = i * tblk + j
        acc2 = jnp.where(tgt == t, h2, acc)
        return h2, acc2

    nsteps = jnp.clip(ml_ref[0] - i * tblk, 0, tblk)
    h, acc = lax.fori_loop(0, nsteps, step, (h_sc[...], out_ref[...]))
    h_sc[...] = h
    out_ref[...] = acc


def _gru_scan(gi, Wh, bh, length, max_len, tblk=256):
    N, B, K = gi.shape
    H = Wh.shape[0]
    nblk = N // tblk
    body = functools.partial(_gru_body, tblk=tblk, H=H, S=K // 3)
    return pl.pallas_call(
        body,
        grid=(nblk,),
        in_specs=[
            pl.BlockSpec((tblk, B, K), lambda i: (i, 0, 0)),
            pl.BlockSpec((H, K), lambda i: (0, 0)),
            pl.BlockSpec((1, K), lambda i: (0, 0)),
            pl.BlockSpec((B, 1), lambda i: (0, 0)),
            pl.BlockSpec(memory_space=pltpu.SMEM),
        ],
        out_specs=pl.BlockSpec((B, H), lambda i: (0, 0)),
        out_shape=jax.ShapeDtypeStruct((B, H), jnp.float32),
        scratch_shapes=[pltpu.VMEM((B, H), jnp.float32)],
    )(gi, Wh, bh, length, max_len)


def kernel(raw_0, raw_1, W0, b0, W1, b1, Wi, Wh, bi, bh, time_index,
           seq_lens_0, seq_lens_1):
    B, L0, F = raw_0.shape
    _, L1, _ = raw_1.shape
    N = time_index.shape[1]
    H = Wh.shape[0]
    S = 128
    KP = 3 * S

    def pad_gates(w):
        parts = jnp.split(w, 3, axis=-1)
        pad = [(0, 0)] * (w.ndim - 1) + [(0, S - H)]
        return jnp.concatenate([jnp.pad(p, pad) for p in parts], axis=-1)

    G = pad_gates(jnp.stack([W0 @ Wi, W1 @ Wi]))
    c = pad_gates(jnp.stack([b0 @ Wi, b1 @ Wi]) + bi)[:, None, :]

    raw_all = jnp.concatenate(
        [raw_0.reshape(B * L0, F), raw_1.reshape(B * L1, F)], axis=0)
    gi_all = _project(raw_all, G, c)

    e_flat = time_index[:, :, 0].reshape(-1)
    m_flat = time_index[:, :, 1].reshape(-1)
    merge = _make_merge(B, N, L0, L1, KP)
    gi_t = merge(gi_all, e_flat, m_flat).reshape(N, B, KP)

    length = (seq_lens_0 + seq_lens_1).astype(jnp.int32).reshape(B, 1)
    max_len = jnp.max(length).reshape(1)
    out = _gru_scan(gi_t, pad_gates(Wh), pad_gates(bh).reshape(1, KP), length,
                    max_len)
    return out


# speedup vs baseline: 1.1970x; 1.0255x over previous
import functools

import jax
import jax.numpy as jnp
from jax import lax
from jax.experimental import pallas as pl
from jax.experimental.pallas import tpu as pltpu
from jax.experimental.pallas import tpu_sc as plsc


def _proj_body(raw_ref, g_ref, c_ref, out_ref):
    out_ref[...] = (
        jnp.dot(raw_ref[...], g_ref[0], preferred_element_type=jnp.float32,
                precision=jax.lax.Precision.DEFAULT)
        + c_ref[0]
    )


def _project(raw_all, G, c, block_rows=1024):
    rows, f = raw_all.shape
    k = G.shape[2]
    n_blocks = rows // block_rows
    per_mod = n_blocks // 2
    return pl.pallas_call(
        _proj_body,
        grid=(n_blocks,),
        in_specs=[
            pl.BlockSpec((block_rows, f), lambda i: (i, 0)),
            pl.BlockSpec((1, f, k), lambda i: (i // per_mod, 0, 0)),
            pl.BlockSpec((1, 1, k), lambda i: (i // per_mod, 0, 0)),
        ],
        out_specs=pl.BlockSpec((block_rows, k), lambda i: (i, 0)),
        out_shape=jax.ShapeDtypeStruct((rows, k), jnp.float32),
    )(raw_all, G, c)


def _make_merge(B, N, L0, L1, K, chunk=128):
    NC, NS = 2, 16
    NW = NC * NS
    wpb = NW // B
    n_per_w = N // wpb
    n_chunks = n_per_w // chunk
    M0 = B * L0
    mesh = plsc.VectorSubcoreMesh(
        core_axis_name="c", subcore_axis_name="s",
        num_cores=NC, num_subcores=NS)

    @functools.partial(
        pl.kernel,
        mesh=mesh,
        out_type=jax.ShapeDtypeStruct((N * B, K), jnp.float32),
        scratch_types=[
            pltpu.VMEM((chunk,), jnp.int32),
            pltpu.VMEM((chunk,), jnp.int32),
            pltpu.VMEM((chunk,), jnp.int32),
            pltpu.VMEM((chunk,), jnp.int32),
            pltpu.VMEM((chunk, K), jnp.float32),
            pltpu.SemaphoreType.DMA,
            pltpu.SemaphoreType.DMA,
        ],
    )
    def merge(table_hbm, e_hbm, m_hbm, out_hbm,
              e_v, m_v, idx_v, oidx_v, rows_v, sem_g, sem_s):
        wid = lax.axis_index("c") * NS + lax.axis_index("s")
        b = wid // wpb
        q = wid % wpb
        iota = lax.iota(jnp.int32, 16)
        for ci in range(n_chunks):
            n0 = q * n_per_w + ci * chunk
            pltpu.sync_copy(e_hbm.at[pl.ds(b * N + n0, chunk)], e_v)
            pltpu.sync_copy(m_hbm.at[pl.ds(b * N + n0, chunk)], m_v)
            for j in range(chunk // 16):
                sl = pl.ds(j * 16, 16)
                ev = e_v[sl]
                mv = m_v[sl]
                idx_v[sl] = mv * M0 + b * L0 + ev
                oidx_v[sl] = (n0 + j * 16 + iota) * B + b
            pltpu.async_copy(table_hbm.at[idx_v], rows_v, sem_g).wait()
            pltpu.async_copy(rows_v, out_hbm.at[oidx_v], sem_s).wait()

    return merge


def _gru_body(gi_ref, wh_ref, bh_ref, len_ref, ml_ref, out_ref, h_sc,
              *, tblk, H, S):
    i = pl.program_id(0)

    @pl.when(i == 0)
    def _init():
        h_sc[...] = jnp.zeros_like(h_sc)
        out_ref[...] = jnp.zeros_like(out_ref)

    wh = wh_ref[...]
    bh = bh_ref[...]
    tgt = len_ref[...] - 1

    def step(j, carry):
        h, acc = carry
        gi = gi_ref[j]
        gh = jnp.dot(h, wh, preferred_element_type=jnp.float32,
                     precision=jax.lax.Precision.DEFAULT) + bh
        r = 0.5 + 0.5 * jnp.tanh(0.5 * (gi[:, 0:H] + gh[:, 0:H]))
        z = 0.5 + 0.5 * jnp.tanh(0.5 * (gi[:, S:S + H] + gh[:, S:S + H]))
        n = jnp.tanh(gi[:, 2 * S:2 * S + H] + r * gh[:, 2 * S:2 * S + H])
        h2 = n + z * (h - n)
        t = i * tblk + j
        acc2 = jnp.where(tgt == t, h2, acc)
        return h2, acc2

    nsteps = jnp.clip(ml_ref[0] - i * tblk, 0, tblk)
    h, acc = lax.fori_loop(0, nsteps, step, (h_sc[...], out_ref[...]))
    h_sc[...] = h
    out_ref[...] = acc


def _gru_scan(gi, Wh, bh, length, max_len, tblk=256):
    N, B, K = gi.shape
    H = Wh.shape[0]
    nblk = N // tblk
    body = functools.partial(_gru_body, tblk=tblk, H=H, S=K // 3)
    return pl.pallas_call(
        body,
        grid=(nblk,),
        in_specs=[
            pl.BlockSpec((tblk, B, K), lambda i: (i, 0, 0)),
            pl.BlockSpec((H, K), lambda i: (0, 0)),
            pl.BlockSpec((1, K), lambda i: (0, 0)),
            pl.BlockSpec((B, 1), lambda i: (0, 0)),
            pl.BlockSpec(memory_space=pltpu.SMEM),
        ],
        out_specs=pl.BlockSpec((B, H), lambda i: (0, 0)),
        out_shape=jax.ShapeDtypeStruct((B, H), jnp.float32),
        scratch_shapes=[pltpu.VMEM((B, H), jnp.float32)],
    )(gi, Wh, bh, length, max_len)


def kernel(raw_0, raw_1, W0, b0, W1, b1, Wi, Wh, bi, bh, time_index,
           seq_lens_0, seq_lens_1):
    B, L0, F = raw_0.shape
    _, L1, _ = raw_1.shape
    N = time_index.shape[1]
    H = Wh.shape[0]
    S = 128
    KP = 3 * S

    def pad_gates(w):
        parts = jnp.split(w, 3, axis=-1)
        pad = [(0, 0)] * (w.ndim - 1) + [(0, S - H)]
        return jnp.concatenate([jnp.pad(p, pad) for p in parts], axis=-1)

    G = pad_gates(jnp.stack([W0 @ Wi, W1 @ Wi]))
    c = pad_gates(jnp.stack([b0 @ Wi, b1 @ Wi]) + bi)[:, None, :]

    raw_all = jnp.concatenate(
        [raw_0.reshape(B * L0, F), raw_1.reshape(B * L1, F)], axis=0)
    gi_all = _project(raw_all, G, c)

    e_flat = time_index[:, :, 0].reshape(-1)
    m_flat = time_index[:, :, 1].reshape(-1)
    merge = _make_merge(B, N, L0, L1, KP)
    gi_t = merge(gi_all, e_flat, m_flat).reshape(N, B, KP)

    length = (seq_lens_0 + seq_lens_1).astype(jnp.int32).reshape(B, 1)
    max_len = jnp.max(length).reshape(1)
    out = _gru_scan(gi_t, pad_gates(Wh), pad_gates(bh).reshape(1, KP), length,
                    max_len)
    return out


# speedup vs baseline: 1.1983x; 1.0011x over previous
import functools

import jax
import jax.numpy as jnp
from jax import lax
from jax.experimental import pallas as pl
from jax.experimental.pallas import tpu as pltpu
from jax.experimental.pallas import tpu_sc as plsc


def _proj_body(raw_ref, g_ref, c_ref, out_ref):
    out_ref[...] = (
        jnp.dot(raw_ref[...], g_ref[0], preferred_element_type=jnp.float32,
                precision=jax.lax.Precision.DEFAULT)
        + c_ref[0]
    )


def _project(raw_all, G, c, block_rows=1024):
    rows, f = raw_all.shape
    k = G.shape[2]
    n_blocks = rows // block_rows
    per_mod = n_blocks // 2
    return pl.pallas_call(
        _proj_body,
        grid=(n_blocks,),
        in_specs=[
            pl.BlockSpec((block_rows, f), lambda i: (i, 0)),
            pl.BlockSpec((1, f, k), lambda i: (i // per_mod, 0, 0)),
            pl.BlockSpec((1, 1, k), lambda i: (i // per_mod, 0, 0)),
        ],
        out_specs=pl.BlockSpec((block_rows, k), lambda i: (i, 0)),
        out_shape=jax.ShapeDtypeStruct((rows, k), jnp.float32),
    )(raw_all, G, c)


def _make_merge(B, N, L0, L1, K, chunk=128):
    NC, NS = 2, 16
    NW = NC * NS
    wpb = NW // B
    n_per_w = N // wpb
    n_chunks = n_per_w // chunk
    M0 = B * L0
    mesh = plsc.VectorSubcoreMesh(
        core_axis_name="c", subcore_axis_name="s",
        num_cores=NC, num_subcores=NS)

    @functools.partial(
        pl.kernel,
        mesh=mesh,
        out_type=jax.ShapeDtypeStruct((N * B, K), jnp.float32),
        scratch_types=[
            pltpu.VMEM((chunk,), jnp.int32),
            pltpu.VMEM((chunk,), jnp.int32),
            pltpu.VMEM((chunk,), jnp.int32),
            pltpu.VMEM((chunk,), jnp.int32),
            pltpu.VMEM((chunk, K), jnp.float32),
            pltpu.SemaphoreType.DMA,
            pltpu.SemaphoreType.DMA,
        ],
    )
    def merge(table_hbm, e_hbm, m_hbm, out_hbm,
              e_v, m_v, idx_v, oidx_v, rows_v, sem_g, sem_s):
        wid = lax.axis_index("c") * NS + lax.axis_index("s")
        b = wid // wpb
        q = wid % wpb
        iota = lax.iota(jnp.int32, 16)
        for ci in range(n_chunks):
            n0 = q * n_per_w + ci * chunk
            pltpu.sync_copy(e_hbm.at[pl.ds(b * N + n0, chunk)], e_v)
            pltpu.sync_copy(m_hbm.at[pl.ds(b * N + n0, chunk)], m_v)
            for j in range(chunk // 16):
                sl = pl.ds(j * 16, 16)
                ev = e_v[sl]
                mv = m_v[sl]
                idx_v[sl] = mv * M0 + b * L0 + ev
                oidx_v[sl] = (n0 + j * 16 + iota) * B + b
            pltpu.async_copy(table_hbm.at[idx_v], rows_v, sem_g).wait()
            pltpu.async_copy(rows_v, out_hbm.at[oidx_v], sem_s).wait()

    return merge


def _gru_body(gi_ref, wh_ref, bh_ref, len_ref, ml_ref, out_ref, h_sc,
              *, tblk, H, S):
    i = pl.program_id(0)

    @pl.when(i == 0)
    def _init():
        h_sc[...] = jnp.zeros_like(h_sc)
        out_ref[...] = jnp.zeros_like(out_ref)

    wh = wh_ref[...].astype(jnp.bfloat16)
    bh = bh_ref[...]
    tgt = len_ref[...] - 1

    def step(j, carry):
        h, acc = carry
        gi = gi_ref[j]
        gh = jnp.dot(h.astype(jnp.bfloat16), wh,
                     preferred_element_type=jnp.float32) + bh
        r = 0.5 + 0.5 * jnp.tanh(0.5 * (gi[:, 0:H] + gh[:, 0:H]))
        z = 0.5 + 0.5 * jnp.tanh(0.5 * (gi[:, S:S + H] + gh[:, S:S + H]))
        n = jnp.tanh(gi[:, 2 * S:2 * S + H] + r * gh[:, 2 * S:2 * S + H])
        h2 = n + z * (h - n)
        t = i * tblk + j
        acc2 = jnp.where(tgt == t, h2, acc)
        return h2, acc2

    nsteps = jnp.clip(ml_ref[0] - i * tblk, 0, tblk)
    h, acc = lax.fori_loop(0, nsteps, step, (h_sc[...], out_ref[...]))
    h_sc[...] = h
    out_ref[...] = acc


def _gru_scan(gi, Wh, bh, length, max_len, tblk=256):
    N, B, K = gi.shape
    H = Wh.shape[0]
    nblk = N // tblk
    body = functools.partial(_gru_body, tblk=tblk, H=H, S=K // 3)
    return pl.pallas_call(
        body,
        grid=(nblk,),
        in_specs=[
            pl.BlockSpec((tblk, B, K), lambda i: (i, 0, 0)),
            pl.BlockSpec((H, K), lambda i: (0, 0)),
            pl.BlockSpec((1, K), lambda i: (0, 0)),
            pl.BlockSpec((B, 1), lambda i: (0, 0)),
            pl.BlockSpec(memory_space=pltpu.SMEM),
        ],
        out_specs=pl.BlockSpec((B, H), lambda i: (0, 0)),
        out_shape=jax.ShapeDtypeStruct((B, H), jnp.float32),
        scratch_shapes=[pltpu.VMEM((B, H), jnp.float32)],
    )(gi, Wh, bh, length, max_len)


def kernel(raw_0, raw_1, W0, b0, W1, b1, Wi, Wh, bi, bh, time_index,
           seq_lens_0, seq_lens_1):
    B, L0, F = raw_0.shape
    _, L1, _ = raw_1.shape
    N = time_index.shape[1]
    H = Wh.shape[0]
    S = 128
    KP = 3 * S

    def pad_gates(w):
        parts = jnp.split(w, 3, axis=-1)
        pad = [(0, 0)] * (w.ndim - 1) + [(0, S - H)]
        return jnp.concatenate([jnp.pad(p, pad) for p in parts], axis=-1)

    G = pad_gates(jnp.stack([W0 @ Wi, W1 @ Wi]))
    c = pad_gates(jnp.stack([b0 @ Wi, b1 @ Wi]) + bi)[:, None, :]

    raw_all = jnp.concatenate(
        [raw_0.reshape(B * L0, F), raw_1.reshape(B * L1, F)], axis=0)
    gi_all = _project(raw_all, G, c)

    e_flat = time_index[:, :, 0].reshape(-1)
    m_flat = time_index[:, :, 1].reshape(-1)
    merge = _make_merge(B, N, L0, L1, KP)
    gi_t = merge(gi_all, e_flat, m_flat).reshape(N, B, KP)

    length = (seq_lens_0 + seq_lens_1).astype(jnp.int32).reshape(B, 1)
    max_len = jnp.max(length).reshape(1)
    out = _gru_scan(gi_t, pad_gates(Wh), pad_gates(bh).reshape(1, KP), length,
                    max_len)
    return out


# speedup vs baseline: 1.2156x; 1.0144x over previous
import functools

import jax
import jax.numpy as jnp
from jax import lax
from jax.experimental import pallas as pl
from jax.experimental.pallas import tpu as pltpu
from jax.experimental.pallas import tpu_sc as plsc


def _proj_body(raw_ref, g_ref, c_ref, out_ref):
    out_ref[...] = (
        jnp.dot(raw_ref[...], g_ref[0], preferred_element_type=jnp.float32,
                precision=jax.lax.Precision.DEFAULT)
        + c_ref[0]
    )


def _project(raw_all, G, c, block_rows=1024):
    rows, f = raw_all.shape
    k = G.shape[2]
    n_blocks = rows // block_rows
    per_mod = n_blocks // 2
    return pl.pallas_call(
        _proj_body,
        grid=(n_blocks,),
        in_specs=[
            pl.BlockSpec((block_rows, f), lambda i: (i, 0)),
            pl.BlockSpec((1, f, k), lambda i: (i // per_mod, 0, 0)),
            pl.BlockSpec((1, 1, k), lambda i: (i // per_mod, 0, 0)),
        ],
        out_specs=pl.BlockSpec((block_rows, k), lambda i: (i, 0)),
        out_shape=jax.ShapeDtypeStruct((rows, k), jnp.float32),
    )(raw_all, G, c)


def _make_merge(B, N, L0, L1, K, chunk=128):
    NC, NS = 2, 16
    NW = NC * NS
    wpb = NW // B
    n_per_w = N // wpb
    n_chunks = n_per_w // chunk
    M0 = B * L0
    mesh = plsc.VectorSubcoreMesh(
        core_axis_name="c", subcore_axis_name="s",
        num_cores=NC, num_subcores=NS)

    @functools.partial(
        pl.kernel,
        mesh=mesh,
        out_type=jax.ShapeDtypeStruct((N * B, K), jnp.float32),
        scratch_types=[
            pltpu.VMEM((n_per_w,), jnp.int32),
            pltpu.VMEM((n_per_w,), jnp.int32),
            pltpu.VMEM((n_chunks, chunk), jnp.int32),
            pltpu.VMEM((n_chunks, chunk), jnp.int32),
            pltpu.VMEM((chunk, K), jnp.float32),
            pltpu.VMEM((chunk, K), jnp.float32),
            pltpu.SemaphoreType.DMA,
            pltpu.SemaphoreType.DMA,
        ],
    )
    def merge(table_hbm, e_hbm, m_hbm, out_hbm,
              e_v, m_v, idx_v, oidx_v, rows_a, rows_b, sem_g, sem_s):
        wid = lax.axis_index("c") * NS + lax.axis_index("s")
        b = wid // wpb
        q = wid % wpb
        base_n = q * n_per_w
        iota = lax.iota(jnp.int32, 16)
        pltpu.sync_copy(e_hbm.at[pl.ds(b * N + base_n, n_per_w)], e_v)
        pltpu.sync_copy(m_hbm.at[pl.ds(b * N + base_n, n_per_w)], m_v)
        for ci in range(n_chunks):
            for j in range(chunk // 16):
                sl = pl.ds(ci * chunk + j * 16, 16)
                ev = e_v[sl]
                mv = m_v[sl]
                dst = pl.ds(j * 16, 16)
                idx_v[ci, dst] = mv * M0 + b * L0 + ev
                oidx_v[ci, dst] = (base_n + ci * chunk + j * 16 + iota) * B + b
        bufs = (rows_a, rows_b)
        pltpu.async_copy(table_hbm.at[idx_v.at[0]], rows_a, sem_g)
        for ci in range(n_chunks):
            cur = bufs[ci % 2]
            pltpu.make_async_copy(table_hbm.at[idx_v.at[ci]], cur, sem_g).wait()
            if ci + 1 < n_chunks:
                pltpu.async_copy(
                    table_hbm.at[idx_v.at[ci + 1]], bufs[(ci + 1) % 2], sem_g)
            pltpu.async_copy(cur, out_hbm.at[oidx_v.at[ci]], sem_s).wait()

    return merge


def _gru_body(gi_ref, wh_ref, bh_ref, len_ref, ml_ref, out_ref, h_sc,
              *, tblk, H, S):
    i = pl.program_id(0)

    @pl.when(i == 0)
    def _init():
        h_sc[...] = jnp.zeros_like(h_sc)
        out_ref[...] = jnp.zeros_like(out_ref)

    wh = wh_ref[...].astype(jnp.bfloat16)
    bh = bh_ref[...]
    Sh = wh.shape[-1] // 3
    tgt = len_ref[...] - 1

    def step(j, carry):
        h, acc = carry
        gi = gi_ref[j]
        gh = jnp.dot(h.astype(jnp.bfloat16), wh,
                     preferred_element_type=jnp.float32) + bh
        r = 0.5 + 0.5 * jnp.tanh(0.5 * (gi[:, 0:H] + gh[:, 0:H]))
        z = 0.5 + 0.5 * jnp.tanh(0.5 * (gi[:, S:S + H] + gh[:, Sh:Sh + H]))
        n = jnp.tanh(gi[:, 2 * S:2 * S + H] + r * gh[:, 2 * Sh:2 * Sh + H])

        h2 = n + z * (h - n)
        t = i * tblk + j
        acc2 = jnp.where(tgt == t, h2, acc)
        return h2, acc2

    nsteps = jnp.clip(ml_ref[0] - i * tblk, 0, tblk)
    h, acc = lax.fori_loop(0, nsteps, step, (h_sc[...], out_ref[...]))
    h_sc[...] = h
    out_ref[...] = acc


def _gru_scan(gi, Wh, bh, length, max_len, tblk=256):
    N, B, K = gi.shape
    H = Wh.shape[0]
    nblk = N // tblk
    body = functools.partial(_gru_body, tblk=tblk, H=H, S=K // 3)
    return pl.pallas_call(
        body,
        grid=(nblk,),
        in_specs=[
            pl.BlockSpec((tblk, B, K), lambda i: (i, 0, 0)),
            pl.BlockSpec((H, K), lambda i: (0, 0)),
            pl.BlockSpec((1, K), lambda i: (0, 0)),
            pl.BlockSpec((B, 1), lambda i: (0, 0)),
            pl.BlockSpec(memory_space=pltpu.SMEM),
        ],
        out_specs=pl.BlockSpec((B, H), lambda i: (0, 0)),
        out_shape=jax.ShapeDtypeStruct((B, H), jnp.float32),
        scratch_shapes=[pltpu.VMEM((B, H), jnp.float32)],
    )(gi, Wh, bh, length, max_len)


def kernel(raw_0, raw_1, W0, b0, W1, b1, Wi, Wh, bi, bh, time_index,
           seq_lens_0, seq_lens_1):
    B, L0, F = raw_0.shape
    _, L1, _ = raw_1.shape
    N = time_index.shape[1]
    H = Wh.shape[0]
    S = 128
    KP = 3 * S

    def pad_gates(w):
        parts = jnp.split(w, 3, axis=-1)
        pad = [(0, 0)] * (w.ndim - 1) + [(0, S - H)]
        return jnp.concatenate([jnp.pad(p, pad) for p in parts], axis=-1)

    G = pad_gates(jnp.stack([W0 @ Wi, W1 @ Wi]))
    c = pad_gates(jnp.stack([b0 @ Wi, b1 @ Wi]) + bi)[:, None, :]

    raw_all = jnp.concatenate(
        [raw_0.reshape(B * L0, F), raw_1.reshape(B * L1, F)], axis=0)
    gi_all = _project(raw_all, G, c)

    e_flat = time_index[:, :, 0].reshape(-1)
    m_flat = time_index[:, :, 1].reshape(-1)
    merge = _make_merge(B, N, L0, L1, KP)
    gi_t = merge(gi_all, e_flat, m_flat).reshape(N, B, KP)

    length = (seq_lens_0 + seq_lens_1).astype(jnp.int32).reshape(B, 1)
    max_len = jnp.max(length).reshape(1)
    out = _gru_scan(gi_t, pad_gates(Wh), pad_gates(bh).reshape(1, KP), length,
                    max_len)
    return out


# speedup vs baseline: 1.2574x; 1.0344x over previous
import functools

import jax
import jax.numpy as jnp
from jax import lax
from jax.experimental import pallas as pl
from jax.experimental.pallas import tpu as pltpu
from jax.experimental.pallas import tpu_sc as plsc


def _proj_body(raw_ref, g_ref, c_ref, out_ref):
    out_ref[...] = (
        jnp.dot(raw_ref[...], g_ref[0], preferred_element_type=jnp.float32,
                precision=jax.lax.Precision.DEFAULT)
        + c_ref[0]
    )


def _project(raw_all, G, c, block_rows=1024):
    rows, f = raw_all.shape
    k = G.shape[2]
    n_blocks = rows // block_rows
    per_mod = n_blocks // 2
    return pl.pallas_call(
        _proj_body,
        grid=(n_blocks,),
        in_specs=[
            pl.BlockSpec((block_rows, f), lambda i: (i, 0)),
            pl.BlockSpec((1, f, k), lambda i: (i // per_mod, 0, 0)),
            pl.BlockSpec((1, 1, k), lambda i: (i // per_mod, 0, 0)),
        ],
        out_specs=pl.BlockSpec((block_rows, k), lambda i: (i, 0)),
        out_shape=jax.ShapeDtypeStruct((rows, k), jnp.float32),
    )(raw_all, G, c)


def _make_merge(B, N, L0, L1, K, chunk=128):
    NC, NS = 2, 16
    NW = NC * NS
    wpb = NW // B
    n_per_w = N // wpb
    n_chunks = n_per_w // chunk
    M0 = B * L0
    mesh = plsc.VectorSubcoreMesh(
        core_axis_name="c", subcore_axis_name="s",
        num_cores=NC, num_subcores=NS)

    @functools.partial(
        pl.kernel,
        mesh=mesh,
        out_type=jax.ShapeDtypeStruct((N * B, K), jnp.float32),
        scratch_types=[
            pltpu.VMEM((n_per_w,), jnp.int32),
            pltpu.VMEM((n_per_w,), jnp.int32),
            pltpu.VMEM((n_chunks, chunk), jnp.int32),
            pltpu.VMEM((n_chunks, chunk), jnp.int32),
            pltpu.VMEM((chunk, K), jnp.float32),
            pltpu.VMEM((chunk, K), jnp.float32),
            pltpu.SemaphoreType.DMA,
            pltpu.SemaphoreType.DMA,
        ],
    )
    def merge(table_hbm, e_hbm, m_hbm, out_hbm,
              e_v, m_v, idx_v, oidx_v, rows_a, rows_b, sem_g, sem_s):
        wid = lax.axis_index("c") * NS + lax.axis_index("s")
        b = wid // wpb
        q = wid % wpb
        base_n = q * n_per_w
        iota = lax.iota(jnp.int32, 16)
        pltpu.sync_copy(e_hbm.at[pl.ds(b * N + base_n, n_per_w)], e_v)
        pltpu.sync_copy(m_hbm.at[pl.ds(b * N + base_n, n_per_w)], m_v)
        for ci in range(n_chunks):
            for j in range(chunk // 16):
                sl = pl.ds(ci * chunk + j * 16, 16)
                ev = e_v[sl]
                mv = m_v[sl]
                dst = pl.ds(j * 16, 16)
                idx_v[ci, dst] = mv * M0 + b * L0 + ev
                oidx_v[ci, dst] = (base_n + ci * chunk + j * 16 + iota) * B + b
        bufs = (rows_a, rows_b)
        pltpu.async_copy(table_hbm.at[idx_v.at[0]], rows_a, sem_g)
        for ci in range(n_chunks):
            cur = bufs[ci % 2]
            pltpu.make_async_copy(table_hbm.at[idx_v.at[ci]], cur, sem_g).wait()
            if ci + 1 < n_chunks:
                pltpu.async_copy(
                    table_hbm.at[idx_v.at[ci + 1]], bufs[(ci + 1) % 2], sem_g)
            pltpu.async_copy(cur, out_hbm.at[oidx_v.at[ci]], sem_s).wait()

    return merge


def _gru_body(gi_ref, wh_ref, bh_ref, len_ref, ml_ref, out_ref, h_sc,
              *, tblk, H, S):
    i = pl.program_id(0)

    @pl.when(i == 0)
    def _init():
        h_sc[...] = jnp.zeros_like(h_sc)
        out_ref[...] = jnp.zeros_like(out_ref)

    wh = wh_ref[...].astype(jnp.bfloat16)
    bh = bh_ref[...]
    Sh = wh.shape[-1] // 3
    tgt = len_ref[...] - 1

    def step(j, carry):
        h, acc = carry
        gi = gi_ref[j]
        gh = jnp.dot(h.astype(jnp.bfloat16), wh,
                     preferred_element_type=jnp.float32) + bh
        r = 0.5 + 0.5 * jnp.tanh(0.5 * (gi[:, 0:H] + gh[:, 0:H]))
        z = 0.5 + 0.5 * jnp.tanh(0.5 * (gi[:, S:S + H] + gh[:, Sh:Sh + H]))
        n = jnp.tanh(gi[:, 2 * S:2 * S + H] + r * gh[:, 2 * Sh:2 * Sh + H])

        h2 = n + z * (h - n)
        t = i * tblk + j
        acc2 = jnp.where(tgt == t, h2, acc)
        return h2, acc2

    U = 4

    def group(jj, carry):
        for u in range(U):
            carry = step(jj * U + u, carry)
        return carry

    nsteps = jnp.clip(ml_ref[0] - i * tblk, 0, tblk)
    ngroups = (nsteps + U - 1) // U
    h, acc = lax.fori_loop(0, ngroups, group, (h_sc[...], out_ref[...]))
    h_sc[...] = h
    out_ref[...] = acc


def _gru_scan(gi, Wh, bh, length, max_len, tblk=256):
    N, B, K = gi.shape
    H = Wh.shape[0]
    nblk = N // tblk
    body = functools.partial(_gru_body, tblk=tblk, H=H, S=K // 3)
    return pl.pallas_call(
        body,
        grid=(nblk,),
        in_specs=[
            pl.BlockSpec((tblk, B, K), lambda i: (i, 0, 0)),
            pl.BlockSpec((H, K), lambda i: (0, 0)),
            pl.BlockSpec((1, K), lambda i: (0, 0)),
            pl.BlockSpec((B, 1), lambda i: (0, 0)),
            pl.BlockSpec(memory_space=pltpu.SMEM),
        ],
        out_specs=pl.BlockSpec((B, H), lambda i: (0, 0)),
        out_shape=jax.ShapeDtypeStruct((B, H), jnp.float32),
        scratch_shapes=[pltpu.VMEM((B, H), jnp.float32)],
    )(gi, Wh, bh, length, max_len)


def kernel(raw_0, raw_1, W0, b0, W1, b1, Wi, Wh, bi, bh, time_index,
           seq_lens_0, seq_lens_1):
    B, L0, F = raw_0.shape
    _, L1, _ = raw_1.shape
    N = time_index.shape[1]
    H = Wh.shape[0]
    S = 128
    KP = 3 * S

    def pad_gates(w):
        parts = jnp.split(w, 3, axis=-1)
        pad = [(0, 0)] * (w.ndim - 1) + [(0, S - H)]
        return jnp.concatenate([jnp.pad(p, pad) for p in parts], axis=-1)

    G = pad_gates(jnp.stack([W0 @ Wi, W1 @ Wi]))
    c = pad_gates(jnp.stack([b0 @ Wi, b1 @ Wi]) + bi)[:, None, :]

    raw_all = jnp.concatenate(
        [raw_0.reshape(B * L0, F), raw_1.reshape(B * L1, F)], axis=0)
    gi_all = _project(raw_all, G, c)

    e_flat = time_index[:, :, 0].reshape(-1)
    m_flat = time_index[:, :, 1].reshape(-1)
    merge = _make_merge(B, N, L0, L1, KP)
    gi_t = merge(gi_all, e_flat, m_flat).reshape(N, B, KP)

    length = (seq_lens_0 + seq_lens_1).astype(jnp.int32).reshape(B, 1)
    max_len = jnp.max(length).reshape(1)
    out = _gru_scan(gi_t, pad_gates(Wh), pad_gates(bh).reshape(1, KP), length,
                    max_len)
    return out


# speedup vs baseline: 1.2804x; 1.0183x over previous
import functools

import jax
import jax.numpy as jnp
from jax import lax
from jax.experimental import pallas as pl
from jax.experimental.pallas import tpu as pltpu
from jax.experimental.pallas import tpu_sc as plsc


def _proj_body(raw_ref, g_ref, c_ref, out_ref):
    out_ref[...] = (
        jnp.dot(raw_ref[...], g_ref[0], preferred_element_type=jnp.float32,
                precision=jax.lax.Precision.DEFAULT)
        + c_ref[0]
    )


def _project(raw_all, G, c, block_rows=1024):
    rows, f = raw_all.shape
    k = G.shape[2]
    n_blocks = rows // block_rows
    per_mod = n_blocks // 2
    return pl.pallas_call(
        _proj_body,
        grid=(n_blocks,),
        in_specs=[
            pl.BlockSpec((block_rows, f), lambda i: (i, 0)),
            pl.BlockSpec((1, f, k), lambda i: (i // per_mod, 0, 0)),
            pl.BlockSpec((1, 1, k), lambda i: (i // per_mod, 0, 0)),
        ],
        out_specs=pl.BlockSpec((block_rows, k), lambda i: (i, 0)),
        out_shape=jax.ShapeDtypeStruct((rows, k), jnp.float32),
    )(raw_all, G, c)


def _make_merge(B, N, L0, L1, K, chunk=128):
    NC, NS = 2, 16
    NW = NC * NS
    wpb = NW // B
    n_per_w = N // wpb
    n_chunks = n_per_w // chunk
    M0 = B * L0
    mesh = plsc.VectorSubcoreMesh(
        core_axis_name="c", subcore_axis_name="s",
        num_cores=NC, num_subcores=NS)

    @functools.partial(
        pl.kernel,
        mesh=mesh,
        out_type=jax.ShapeDtypeStruct((N * B, K), jnp.float32),
        scratch_types=[
            pltpu.VMEM((n_per_w,), jnp.int32),
            pltpu.VMEM((n_per_w,), jnp.int32),
            pltpu.VMEM((n_chunks, chunk), jnp.int32),
            pltpu.VMEM((n_chunks, chunk), jnp.int32),
            pltpu.VMEM((chunk, K), jnp.float32),
            pltpu.VMEM((chunk, K), jnp.float32),
            pltpu.SemaphoreType.DMA,
            pltpu.SemaphoreType.DMA,
        ],
    )
    def merge(table_hbm, e_hbm, m_hbm, out_hbm,
              e_v, m_v, idx_v, oidx_v, rows_a, rows_b, sem_g, sem_s):
        wid = lax.axis_index("c") * NS + lax.axis_index("s")
        b = wid // wpb
        q = wid % wpb
        base_n = q * n_per_w
        iota = lax.iota(jnp.int32, 16)
        pltpu.sync_copy(e_hbm.at[pl.ds(b * N + base_n, n_per_w)], e_v)
        pltpu.sync_copy(m_hbm.at[pl.ds(b * N + base_n, n_per_w)], m_v)
        for ci in range(n_chunks):
            for j in range(chunk // 16):
                sl = pl.ds(ci * chunk + j * 16, 16)
                ev = e_v[sl]
                mv = m_v[sl]
                dst = pl.ds(j * 16, 16)
                idx_v[ci, dst] = mv * M0 + b * L0 + ev
                oidx_v[ci, dst] = (base_n + ci * chunk + j * 16 + iota) * B + b
        bufs = (rows_a, rows_b)
        pltpu.async_copy(table_hbm.at[idx_v.at[0]], rows_a, sem_g)
        for ci in range(n_chunks):
            cur = bufs[ci % 2]
            pltpu.make_async_copy(table_hbm.at[idx_v.at[ci]], cur, sem_g).wait()
            if ci + 1 < n_chunks:
                pltpu.async_copy(
                    table_hbm.at[idx_v.at[ci + 1]], bufs[(ci + 1) % 2], sem_g)
            pltpu.async_copy(cur, out_hbm.at[oidx_v.at[ci]], sem_s).wait()

    return merge


def _gru_body(gi_ref, wh_ref, bh_ref, len_ref, ml_ref, out_ref, h_sc,
              *, tblk, H, S):
    i = pl.program_id(0)

    @pl.when(i == 0)
    def _init():
        h_sc[...] = jnp.zeros_like(h_sc)
        out_ref[...] = jnp.zeros_like(out_ref)

    wh = wh_ref[...].astype(jnp.bfloat16)
    bh = bh_ref[...]
    Sh = wh.shape[-1] // 3
    tgt = len_ref[...] - 1

    def step(j, carry):
        h, acc = carry
        gi = gi_ref[j]
        gh = jnp.dot(h.astype(jnp.bfloat16), wh,
                     preferred_element_type=jnp.float32) + bh
        h_n = gh[:, 2 * Sh:2 * Sh + H]
        tau_r = jnp.tanh(0.5 * (gi[:, 0:H] + gh[:, 0:H]))
        tau_z = jnp.tanh(0.5 * (gi[:, S:S + H] + gh[:, Sh:Sh + H]))
        half_hn = 0.5 * h_n
        n_base = gi[:, 2 * S:2 * S + H] + half_hn
        z = 0.5 + 0.5 * tau_z
        zh = z * h
        n = jnp.tanh(n_base + tau_r * half_hn)
        h2 = (1.0 - z) * n + zh
        t = i * tblk + j
        acc2 = jnp.where(tgt == t, h2, acc)
        return h2, acc2

    U = 4

    def group(jj, carry):
        for u in range(U):
            carry = step(jj * U + u, carry)
        return carry

    nsteps = jnp.clip(ml_ref[0] - i * tblk, 0, tblk)
    ngroups = (nsteps + U - 1) // U
    h, acc = lax.fori_loop(0, ngroups, group, (h_sc[...], out_ref[...]))
    h_sc[...] = h
    out_ref[...] = acc


def _gru_scan(gi, Wh, bh, length, max_len, tblk=256):
    N, B, K = gi.shape
    H = Wh.shape[0]
    nblk = N // tblk
    body = functools.partial(_gru_body, tblk=tblk, H=H, S=K // 3)
    return pl.pallas_call(
        body,
        grid=(nblk,),
        in_specs=[
            pl.BlockSpec((tblk, B, K), lambda i: (i, 0, 0)),
            pl.BlockSpec((H, K), lambda i: (0, 0)),
            pl.BlockSpec((1, K), lambda i: (0, 0)),
            pl.BlockSpec((B, 1), lambda i: (0, 0)),
            pl.BlockSpec(memory_space=pltpu.SMEM),
        ],
        out_specs=pl.BlockSpec((B, H), lambda i: (0, 0)),
        out_shape=jax.ShapeDtypeStruct((B, H), jnp.float32),
        scratch_shapes=[pltpu.VMEM((B, H), jnp.float32)],
    )(gi, Wh, bh, length, max_len)


def kernel(raw_0, raw_1, W0, b0, W1, b1, Wi, Wh, bi, bh, time_index,
           seq_lens_0, seq_lens_1):
    B, L0, F = raw_0.shape
    _, L1, _ = raw_1.shape
    N = time_index.shape[1]
    H = Wh.shape[0]
    S = 128
    KP = 3 * S

    def pad_gates(w):
        parts = jnp.split(w, 3, axis=-1)
        pad = [(0, 0)] * (w.ndim - 1) + [(0, S - H)]
        return jnp.concatenate([jnp.pad(p, pad) for p in parts], axis=-1)

    G = pad_gates(jnp.stack([W0 @ Wi, W1 @ Wi]))
    c = pad_gates(jnp.stack([b0 @ Wi, b1 @ Wi]) + bi)[:, None, :]

    raw_all = jnp.concatenate(
        [raw_0.reshape(B * L0, F), raw_1.reshape(B * L1, F)], axis=0)
    gi_all = _project(raw_all, G, c)

    e_flat = time_index[:, :, 0].reshape(-1)
    m_flat = time_index[:, :, 1].reshape(-1)
    merge = _make_merge(B, N, L0, L1, KP)
    gi_t = merge(gi_all, e_flat, m_flat).reshape(N, B, KP)

    length = (seq_lens_0 + seq_lens_1).astype(jnp.int32).reshape(B, 1)
    max_len = jnp.max(length).reshape(1)
    out = _gru_scan(gi_t, pad_gates(Wh), pad_gates(bh).reshape(1, KP), length,
                    max_len)
    return out
